# Initial kernel scaffold; baseline (speedup 1.0000x reference)
import jax
import jax.numpy as jnp
from jax.experimental import pallas as pl


def kernel(x, edge_index, W1, al1, ar1, b1, W2, al2, ar2, b2):
    raise NotImplementedError("write your pallas kernel here")


# speedup vs baseline: 14.3573x; 14.3573x over previous
import functools

import jax
import jax.numpy as jnp
from jax import lax
from jax.experimental import pallas as pl
from jax.experimental.pallas import tpu as pltpu
from jax.experimental.pallas import tpu_sc as plsc

NNODES = 10000
D = 128
NPAD = 10240
NTILES = 32
EB = 128
BR = 512
NEG = -1e30
NROWS_T = NPAD // 16


def _head_math(h, al_ref, ar_ref, w_ref, i):
    feat = jnp.dot(h, w_ref[...], preferred_element_type=jnp.float32)
    el = jnp.sum(feat * al_ref[...], axis=1)
    er = jnp.sum(feat * ar_ref[...], axis=1)
    rows = i * BR + lax.broadcasted_iota(jnp.int32, (BR,), 0)
    valid = rows < NNODES
    el = jnp.where(valid, el, NEG)
    er = jnp.where(valid, er, NEG)
    return feat, el, er


def _head_finish(i, el, er, el_ref, er_ref, m_ref, mx_ref):
    el_ref[...] = el.reshape(1, BR)
    er_ref[...] = er.reshape(1, BR)

    @pl.when(i == 0)
    def _():
        mx_ref[0] = NEG
        mx_ref[1] = NEG

    mx_ref[0] = jnp.maximum(mx_ref[0], jnp.max(el))
    mx_ref[1] = jnp.maximum(mx_ref[1], jnp.max(er))

    @pl.when(i == pl.num_programs(0) - 1)
    def _():
        m = jnp.maximum(mx_ref[0] + mx_ref[1], 0.0)
        m_ref[...] = jnp.full((1, 16), m, jnp.float32)


def _head_body(h_ref, w_ref, al_ref, ar_ref,
               feat_ref, el_ref, er_ref, m_ref, mx_ref):
    i = pl.program_id(0)
    feat, el, er = _head_math(h_ref[...], al_ref, ar_ref, w_ref, i)
    feat_ref[...] = feat
    _head_finish(i, el, er, el_ref, er_ref, m_ref, mx_ref)


def _merge_elu(num_ref, den_ref, b_ref):
    ns = num_ref[0] + num_ref[1]
    dn = den_ref[0, 0] + den_ref[1, 0]
    dn = jnp.maximum(dn, 1e-30)
    h = ns / dn[:, None] + b_ref[...]
    return jnp.where(h > 0, h, jnp.exp(jnp.minimum(h, 0.0)) - 1.0)


def _merge_head_body(num_ref, den_ref, b_ref, w_ref, al_ref, ar_ref,
                     feat_ref, el_ref, er_ref, m_ref, mx_ref):
    i = pl.program_id(0)
    h = _merge_elu(num_ref, den_ref, b_ref)
    feat, el, er = _head_math(h, al_ref, ar_ref, w_ref, i)
    feat_ref[...] = feat
    _head_finish(i, el, er, el_ref, er_ref, m_ref, mx_ref)


def _merge_out_body(num_ref, den_ref, b_ref, out_ref):
    out_ref[...] = _merge_elu(num_ref, den_ref, b_ref)


_HEAD_OUT_SPECS = [
    pl.BlockSpec((BR, D), lambda i: (i, 0)),
    pl.BlockSpec((1, BR), lambda i: (0, i)),
    pl.BlockSpec((1, BR), lambda i: (0, i)),
    pl.BlockSpec((1, 16), lambda i: (0, 0)),
]
_HEAD_OUT_SHAPE = [
    jax.ShapeDtypeStruct((NPAD, D), jnp.float32),
    jax.ShapeDtypeStruct((1, NPAD), jnp.float32),
    jax.ShapeDtypeStruct((1, NPAD), jnp.float32),
    jax.ShapeDtypeStruct((1, 16), jnp.float32),
]
_FULL = lambda i: (0, 0)


def _head(h, W, al, ar):
    return pl.pallas_call(
        _head_body,
        grid=(NPAD // BR,),
        in_specs=[
            pl.BlockSpec((BR, D), lambda i: (i, 0)),
            pl.BlockSpec((D, D), _FULL),
            pl.BlockSpec((1, D), _FULL),
            pl.BlockSpec((1, D), _FULL),
        ],
        out_specs=_HEAD_OUT_SPECS,
        out_shape=_HEAD_OUT_SHAPE,
        scratch_shapes=[pltpu.SMEM((2,), jnp.float32)],
    )(h, W, al, ar)


def _merge_head(num, den, b, W, al, ar):
    return pl.pallas_call(
        _merge_head_body,
        grid=(NPAD // BR,),
        in_specs=[
            pl.BlockSpec((2, BR, D), lambda i: (0, i, 0)),
            pl.BlockSpec((2, 1, BR), lambda i: (0, 0, i)),
            pl.BlockSpec((1, D), _FULL),
            pl.BlockSpec((D, D), _FULL),
            pl.BlockSpec((1, D), _FULL),
            pl.BlockSpec((1, D), _FULL),
        ],
        out_specs=_HEAD_OUT_SPECS,
        out_shape=_HEAD_OUT_SHAPE,
        scratch_shapes=[pltpu.SMEM((2,), jnp.float32)],
    )(num, den, b, W, al, ar)


def _merge_out(num, den, b):
    return pl.pallas_call(
        _merge_out_body,
        grid=(NPAD // BR,),
        in_specs=[
            pl.BlockSpec((2, BR, D), lambda i: (0, i, 0)),
            pl.BlockSpec((2, 1, BR), lambda i: (0, 0, i)),
            pl.BlockSpec((1, D), _FULL),
        ],
        out_specs=pl.BlockSpec((BR, D), lambda i: (i, 0)),
        out_shape=jax.ShapeDtypeStruct((NPAD, D), jnp.float32),
    )(num, den, b)


def _make_sc_edge(nb):
    @functools.partial(
        pl.kernel,
        out_type=(
            jax.ShapeDtypeStruct((2, NPAD, D), jnp.float32),
            jax.ShapeDtypeStruct((2, NPAD), jnp.float32),
        ),
        mesh=plsc.VectorSubcoreMesh(core_axis_name="c", subcore_axis_name="s"),
        compiler_params=pltpu.CompilerParams(needs_layout_passes=False),
        scratch_types=[
            pltpu.VMEM((EB,), jnp.int32),
            pltpu.VMEM((EB,), jnp.int32),
            pltpu.VMEM((EB,), jnp.float32),
            pltpu.VMEM((EB,), jnp.float32),
            pltpu.VMEM((16,), jnp.float32),
            pltpu.VMEM((EB, D), jnp.float32),
            pltpu.VMEM((EB,), jnp.float32),
            pltpu.VMEM_SHARED((NPAD, D), jnp.float32),
            pltpu.VMEM_SHARED((NPAD,), jnp.float32),
        ],
    )
    def sc_edge(feat_hbm, el_hbm, er_hbm, m_hbm, src_hbm, dst_hbm,
                num_hbm, den_hbm,
                src_v, dst_v, elg_v, erg_v, m_v, rows_v, w_v, num_sh, den_sh):
        c = lax.axis_index("c")
        s = lax.axis_index("s")
        g = c * 16 + s
        pltpu.sync_copy(m_hbm, m_v)

        zero16 = jnp.zeros((16,), jnp.float32)

        def _zr(i, carry):
            r = i // 8
            cc = i % 8
            rows_v[r, pl.ds(cc * 16, 16)] = zero16
            return carry

        lax.fori_loop(0, EB * 8, _zr, 0)

        roff = s * NROWS_T
        for k in range(NROWS_T // EB):
            pltpu.sync_copy(rows_v, num_sh.at[pl.ds(roff + k * EB, EB)])
            pltpu.sync_copy(rows_v.at[0], den_sh.at[pl.ds(roff + k * EB, EB)])
        plsc.subcore_barrier()

        m_vec = m_v[...]

        def _block(j, carry):
            pltpu.sync_copy(src_hbm.at[g, j], src_v)
            pltpu.sync_copy(dst_hbm.at[g, j], dst_v)
            pltpu.sync_copy(el_hbm.at[src_v], elg_v)
            pltpu.sync_copy(er_hbm.at[dst_v], erg_v)
            pltpu.sync_copy(feat_hbm.at[src_v], rows_v)

            def _grp(gi, cc):
                sl = pl.ds(gi * 16, 16)
                e = elg_v[sl] + erg_v[sl]
                e = jnp.where(e >= 0, e, 0.2 * e)
                w_v[sl] = jnp.exp(e - m_vec)
                return cc

            lax.fori_loop(0, 8, _grp, 0)

            def _scale(k, cc):
                kidx = jnp.full((16,), k, jnp.int32)
                wvec = plsc.load_gather(w_v, [kidx])
                for q in range(8):
                    sl = pl.ds(q * 16, 16)
                    rows_v[k, sl] = rows_v[k, sl] * wvec
                return cc

            lax.fori_loop(0, EB, _scale, 0)

            pltpu.sync_copy(rows_v, num_sh.at[dst_v], add=True)
            pltpu.sync_copy(w_v, den_sh.at[dst_v], add=True)
            return carry

        lax.fori_loop(0, nb, _block, 0)

        plsc.subcore_barrier()
        pltpu.sync_copy(num_sh.at[pl.ds(roff, NROWS_T)],
                        num_hbm.at[c, pl.ds(roff, NROWS_T)])
        pltpu.sync_copy(den_sh.at[pl.ds(roff, NROWS_T)],
                        den_hbm.at[c, pl.ds(roff, NROWS_T)])

    return sc_edge


def kernel(x, edge_index, W1, al1, ar1, b1, W2, al2, ar2, b2):
    nedges = edge_index.shape[1]
    nb = -(-nedges // (NTILES * EB))
    epad = NTILES * nb * EB
    src = edge_index[0].astype(jnp.int32)
    dst = edge_index[1].astype(jnp.int32)
    pad = jnp.full((epad - nedges,), NNODES, jnp.int32)
    src_t = jnp.concatenate([src, pad]).reshape(NTILES, nb, EB)
    dst_t = jnp.concatenate([dst, pad]).reshape(NTILES, nb, EB)
    xp = jnp.pad(x, ((0, NPAD - x.shape[0]), (0, 0)))

    al1r, ar1r = al1.reshape(1, D), ar1.reshape(1, D)
    al2r, ar2r = al2.reshape(1, D), ar2.reshape(1, D)
    b1r, b2r = b1.reshape(1, D), b2.reshape(1, D)

    sc_edge = _make_sc_edge(nb)

    feat1, el1, er1, m1 = _head(xp, W1, al1r, ar1r)
    num1, den1 = sc_edge(feat1, el1.reshape(NPAD), er1.reshape(NPAD),
                         m1.reshape(16), src_t, dst_t)
    feat2, el2, er2, m2 = _merge_head(num1, den1.reshape(2, 1, NPAD), b1r,
                                      W2, al2r, ar2r)
    num2, den2 = sc_edge(feat2, el2.reshape(NPAD), er2.reshape(NPAD),
                         m2.reshape(16), src_t, dst_t)
    out = _merge_out(num2, den2.reshape(2, 1, NPAD), b2r)
    return out[:NNODES]
